# baseline (speedup 1.0000x reference)
import functools

import jax
import jax.numpy as jnp
from jax import lax
from jax.experimental import pallas as pl
from jax.experimental.pallas import tpu as pltpu
from jax.experimental.pallas import tpu_sc as plsc

NUM_CODEWORDS = 512
CODEWORDS_DIM = 64
COMMITMENT_COST = 0.25
GRID = 2
SC_NC = 2
SC_NS = 16
SC_NW = SC_NC * SC_NS


def _score_kernel(x_ref, cw_ref, idx_ref, loss_ref):
    g = pl.program_id(0)
    B = x_ref.shape[0]
    cw = cw_ref[...]
    cn = jnp.sum(cw * cw, axis=1)
    cw2 = cw * (-2.0)
    x = jnp.concatenate([x_ref[b] for b in range(B)], axis=1)
    prod = jax.lax.dot_general(
        cw2, x, (((1,), (0,)), ((), ())),
        preferred_element_type=jnp.float32,
        precision=jax.lax.Precision.HIGHEST,
    )
    s = cn[:, None] + prod
    idx = jnp.argmin(s, axis=0)
    idx_ref[0, 0] = idx
    scale = (1.0 + COMMITMENT_COST) / (pl.num_programs(0) * x.size)
    part = (jnp.sum(jnp.min(s, axis=0)) + jnp.sum(x * x)) * scale

    @pl.when(g == 0)
    def _init():
        loss_ref[0, 0] = 0.0

    loss_ref[0, 0] += part


def _make_sc_gather(n_points):
    chunk = n_points // SC_NW
    mesh = plsc.VectorSubcoreMesh(core_axis_name="c", subcore_axis_name="s")

    @functools.partial(
        pl.kernel, mesh=mesh,
        compiler_params=pltpu.CompilerParams(use_tc_tiling_on_sc=False),
        out_type=jax.ShapeDtypeStruct((n_points, CODEWORDS_DIM), jnp.float32),
        scratch_types=[
            pltpu.VMEM((chunk,), jnp.int32),
            pltpu.VMEM((chunk, CODEWORDS_DIM), jnp.float32),
            pltpu.SemaphoreType.DMA,
        ],
    )
    def gather(table_hbm, idx_hbm, out_hbm, idx_v, rows_v, sem):
        wid = lax.axis_index("s") * SC_NC + lax.axis_index("c")
        base = wid * chunk
        pltpu.sync_copy(idx_hbm.at[pl.ds(base, chunk)], idx_v)
        pltpu.async_copy(table_hbm.at[idx_v], rows_v, sem).wait()
        pltpu.sync_copy(rows_v, out_hbm.at[pl.ds(base, chunk)])

    return gather


def kernel(inputs, codewords):
    B, C, H, W = inputs.shape
    N = B * H * W
    BG = B // GRID
    x = inputs.reshape(B, C, H * W)
    idx, loss = pl.pallas_call(
        _score_kernel,
        grid=(GRID,),
        in_specs=[
            pl.BlockSpec((BG, C, H * W), lambda g: (g, 0, 0)),
            pl.BlockSpec((NUM_CODEWORDS, C), lambda g: (0, 0)),
        ],
        out_specs=[
            pl.BlockSpec((1, 1, BG * H * W), lambda g: (g, 0, 0)),
            pl.BlockSpec(memory_space=pltpu.SMEM, block_shape=(1, 1),
                         index_map=lambda g: (0, 0)),
        ],
        out_shape=[
            jax.ShapeDtypeStruct((GRID, 1, BG * H * W), jnp.int32),
            jax.ShapeDtypeStruct((1, 1), jnp.float32),
        ],
    )(x, codewords)
    flat_idx = idx.reshape(N)
    gathered = _make_sc_gather(N)(codewords, flat_idx)
    quantized = (gathered.reshape(B, H * W, C)
                 .transpose(0, 2, 1)
                 .reshape(B, C, H, W))
    encoding_indices = idx.reshape(B, H, W)
    return quantized, encoding_indices, loss[0, 0]


# speedup vs baseline: 2.4025x; 2.4025x over previous
import jax
import jax.numpy as jnp
from jax.experimental import pallas as pl
from jax.experimental.pallas import tpu as pltpu

NUM_CODEWORDS = 512
CODEWORDS_DIM = 64
COMMITMENT_COST = 0.25
GRID = 4


def _split2(a):
    a0 = a.astype(jnp.bfloat16)
    a1 = (a - a0.astype(jnp.float32)).astype(jnp.bfloat16)
    return a0, a1


def _vq_kernel(x_ref, cw_ref, q_ref, idx_ref, loss_ref):
    g = pl.program_id(0)
    B = x_ref.shape[0]
    HW = x_ref.shape[2]
    cw = cw_ref[...]
    cn = jnp.sum(cw * cw, axis=1)
    cw2 = cw * (-2.0)
    x = jnp.concatenate([x_ref[b] for b in range(B)], axis=1)
    prod = jax.lax.dot_general(
        cw2, x, (((1,), (0,)), ((), ())),
        preferred_element_type=jnp.float32,
        precision=jax.lax.Precision.HIGHEST,
    )
    s = cn[:, None] + prod
    idx = jnp.argmin(s, axis=0)
    idx_ref[0, 0] = idx
    iota_k = jax.lax.broadcasted_iota(jnp.int32, s.shape, 0)
    onehot = (iota_k == idx[None, :]).astype(jnp.bfloat16)
    c0, c1 = _split2(cw)
    dn = (((0,), (0,)), ((), ()))
    q = (jax.lax.dot_general(c0, onehot, dn, preferred_element_type=jnp.float32)
         + jax.lax.dot_general(c1, onehot, dn, preferred_element_type=jnp.float32))
    for b in range(B):
        q_ref[b] = q[:, b * HW:(b + 1) * HW]
    scale = (1.0 + COMMITMENT_COST) / (pl.num_programs(0) * x.size)
    part = (jnp.sum(jnp.min(s, axis=0)) + jnp.sum(x * x)) * scale

    @pl.when(g == 0)
    def _init():
        loss_ref[0, 0] = 0.0

    loss_ref[0, 0] += part


def kernel(inputs, codewords):
    B, C, H, W = inputs.shape
    N = B * H * W
    BG = B // GRID
    x = inputs.reshape(B, C, H * W)
    q, idx, loss = pl.pallas_call(
        _vq_kernel,
        grid=(GRID,),
        in_specs=[
            pl.BlockSpec((BG, C, H * W), lambda g: (g, 0, 0)),
            pl.BlockSpec((NUM_CODEWORDS, C), lambda g: (0, 0)),
        ],
        out_specs=[
            pl.BlockSpec((BG, C, H * W), lambda g: (g, 0, 0)),
            pl.BlockSpec((1, 1, BG * H * W), lambda g: (g, 0, 0)),
            pl.BlockSpec(memory_space=pltpu.SMEM, block_shape=(1, 1),
                         index_map=lambda g: (0, 0)),
        ],
        out_shape=[
            jax.ShapeDtypeStruct((B, C, H * W), jnp.float32),
            jax.ShapeDtypeStruct((GRID, 1, BG * H * W), jnp.int32),
            jax.ShapeDtypeStruct((1, 1), jnp.float32),
        ],
    )(x, codewords)
    quantized = q.reshape(B, C, H, W)
    encoding_indices = idx.reshape(B, H, W)
    return quantized, encoding_indices, loss[0, 0]
